# baseline (speedup 1.0000x reference)
import functools

import jax
import jax.numpy as jnp
from jax import lax
from jax.experimental import pallas as pl
from jax.experimental.pallas import tpu as pltpu
from jax.experimental.pallas import tpu_sc as plsc

N_OBS = 10000
F = 64
E = 320000
NUM_LAYERS = 3

E_BLK = 5000
NB = E // E_BLK

NC = 2
NS = 16
E_PER_SC = E // NC
E_PER_W = E_PER_SC // NS
CH = 80
NCH = E_PER_W // CH
RPAD = 640
NPAD = NS * RPAD

_f32 = jnp.float32


def _k1_body(ev_ref, dst_ref, psrc_ref, wmf_f_ref,
             wcat_ref, bias_ref, mof_ref, epe_ref, aggf_ref, cnt_ref):
    ev_row = ev_ref[0]
    dst_row = dst_ref[0]
    ep = lax.dot_general(ev_row, wcat_ref[...], (((0,), (0,)), ((), ())),
                         precision=lax.Precision.HIGHEST,
                         preferred_element_type=_f32)
    iota_col = lax.broadcasted_iota(jnp.int32, (F, 1), 0).astype(_f32)
    oht = (iota_col == dst_row).astype(_f32)
    m_of = jnp.maximum(
        lax.dot_general(oht, wmf_f_ref[...], (((0,), (0,)), ((), ())),
                        precision=lax.Precision.HIGHEST,
                        preferred_element_type=_f32)
        + ep[:, :F] + bias_ref[:, :F], 0.0)
    mof_ref[...] = m_of
    m_fo = jnp.maximum(psrc_ref[...] + ep[:, F:2 * F]
                       + bias_ref[:, F:2 * F], 0.0)
    epe_ref[...] = ep[:, 2 * F:]

    @pl.when(pl.program_id(0) == 0)
    def _():
        aggf_ref[...] = jnp.zeros_like(aggf_ref)
        cnt_ref[...] = jnp.zeros_like(cnt_ref)

    aggf_ref[...] += jnp.dot(oht, m_fo, precision=lax.Precision.HIGHEST,
                             preferred_element_type=_f32)
    ones_col = jnp.ones((E_BLK, 1), _f32)
    cnt_ref[...] += jnp.dot(oht, ones_col, precision=lax.Precision.HIGHEST,
                            preferred_element_type=_f32)


def _run_k1(ev3, dst3, psrc, wmf_f, wcat, bias):
    return pl.pallas_call(
        _k1_body,
        grid=(NB,),
        in_specs=[
            pl.BlockSpec((1, 1, E_BLK), lambda i: (i, 0, 0)),
            pl.BlockSpec((1, 1, E_BLK), lambda i: (i, 0, 0)),
            pl.BlockSpec((1, F), lambda i: (0, 0)),
            pl.BlockSpec((F, F), lambda i: (0, 0)),
            pl.BlockSpec((1, 3 * F), lambda i: (0, 0)),
            pl.BlockSpec((1, 3 * F), lambda i: (0, 0)),
        ],
        out_specs=[
            pl.BlockSpec((E_BLK, F), lambda i: (i, 0)),
            pl.BlockSpec((E_BLK, F), lambda i: (i, 0)),
            pl.BlockSpec((F, F), lambda i: (0, 0)),
            pl.BlockSpec((F, 1), lambda i: (0, 0)),
        ],
        out_shape=[
            jax.ShapeDtypeStruct((E, F), _f32),
            jax.ShapeDtypeStruct((E, F), _f32),
            jax.ShapeDtypeStruct((F, F), _f32),
            jax.ShapeDtypeStruct((F, 1), _f32),
        ],
        compiler_params=pltpu.CompilerParams(
            dimension_semantics=("arbitrary",)),
    )(ev3, dst3, psrc, wmf_f, wcat, bias)


def _k3_body(last, node_ref, aggo_ref, cntp_ref, aggf_ref, cntd_ref,
             feat_ref, wn_n_ref, wn_a_ref, bn_ref, wf_f_ref, wf_a_ref,
             bf_ref, we_n_ref, we_f_ref, wmo_nx_ref,
             node2_ref, *out_refs):
    if last:
        feat2_ref, = out_refs
    else:
        a_ref, feat2_ref, b_ref = out_refs
    node = node_ref[...]
    cnt = jnp.maximum(cntp_ref[0][:N_OBS, 0:1] + cntp_ref[1][:N_OBS, 0:1], 1.0)
    aggo = (aggo_ref[0][:N_OBS] + aggo_ref[1][:N_OBS]) / cnt
    new_node = jnp.maximum(
        jnp.dot(node, wn_n_ref[...], preferred_element_type=_f32)
        + jnp.dot(aggo, wn_a_ref[...], preferred_element_type=_f32)
        + bn_ref[...], 0.0)
    node2 = jnp.maximum(new_node + node, 0.0)
    node2_ref[...] = node2
    if not last:
        a = jnp.dot(new_node, we_n_ref[...], preferred_element_type=_f32)
        pn = jnp.dot(node2, wmo_nx_ref[...], preferred_element_type=_f32)
        a_ref[...] = jnp.concatenate([a, pn], axis=1)
    feat = feat_ref[...]
    aggf = aggf_ref[...] / jnp.maximum(cntd_ref[...], 1.0)
    new_feat = jnp.maximum(
        jnp.dot(feat, wf_f_ref[...], preferred_element_type=_f32)
        + jnp.dot(aggf, wf_a_ref[...], preferred_element_type=_f32)
        + bf_ref[...], 0.0)
    feat2 = jnp.maximum(new_feat + feat, 0.0)
    feat2_ref[...] = feat2
    if not last:
        b_ref[...] = jnp.dot(new_feat, we_f_ref[...],
                             preferred_element_type=_f32)


def _run_k3(last, node, aggo_parts, cnt_parts, aggf, cntd, feat,
            wn_n, wn_a, bn, wf_f, wf_a, bf, we_n, we_f, wmo_nx):
    return pl.pallas_call(
        functools.partial(_k3_body, last),
        out_shape=([jax.ShapeDtypeStruct((N_OBS, F), _f32),
                    jax.ShapeDtypeStruct((F, F), _f32)] if last else
                   [jax.ShapeDtypeStruct((N_OBS, F), _f32),
                    jax.ShapeDtypeStruct((N_OBS, 2 * F), _f32),
                    jax.ShapeDtypeStruct((F, F), _f32),
                    jax.ShapeDtypeStruct((F, F), _f32)]),
    )(node, aggo_parts, cnt_parts, aggf, cntd, feat,
      wn_n, wn_a, bn, wf_f, wf_a, bf, we_n, we_f, wmo_nx)


def _kb_body(mid, *refs):
    (epe_ref, g_ref, dst_ref, prev_ref, b0_ref, be0_ref,
     feat_ref, wmf_f_ref, wcat_ref, bias_ref) = refs[:10]
    if mid:
        edge_ref, mof_ref, epe_out_ref, aggf_ref, cnt_ref = refs[10:]
    else:
        mof_ref, aggf_ref, cnt_ref = refs[10:]
    iota_col = lax.broadcasted_iota(jnp.int32, (F, 1), 0).astype(_f32)
    dst_row = dst_ref[0]
    oht = (iota_col == dst_row).astype(_f32)
    g = g_ref[...]
    new_edge = jnp.maximum(
        epe_ref[...] + g[:, :F]
        + lax.dot_general(oht, b0_ref[...], (((0,), (0,)), ((), ())),
                          precision=lax.Precision.HIGHEST,
                          preferred_element_type=_f32)
        + be0_ref[...], 0.0)
    if mid:
        prev = lax.dot_general(prev_ref[0], jnp.ones((1, F), _f32),
                               (((0,), (0,)), ((), ())),
                               precision=lax.Precision.HIGHEST,
                               preferred_element_type=_f32)
    else:
        prev = prev_ref[...]
    edge = jnp.maximum(new_edge + prev, 0.0)
    if mid:
        edge_ref[...] = edge
    ep = jnp.dot(edge, wcat_ref[...], preferred_element_type=_f32)
    t = jnp.dot(feat_ref[...], wmf_f_ref[...], preferred_element_type=_f32)
    m_of = jnp.maximum(
        lax.dot_general(oht, t, (((0,), (0,)), ((), ())),
                        precision=lax.Precision.HIGHEST,
                        preferred_element_type=_f32)
        + ep[:, :F] + bias_ref[:, :F], 0.0)
    mof_ref[...] = m_of
    m_fo = jnp.maximum(g[:, F:2 * F] + ep[:, F:2 * F]
                       + bias_ref[:, F:2 * F], 0.0)
    if mid:
        epe_out_ref[...] = ep[:, 2 * F:]

    @pl.when(pl.program_id(0) == 0)
    def _():
        aggf_ref[...] = jnp.zeros_like(aggf_ref)
        cnt_ref[...] = jnp.zeros_like(cnt_ref)

    aggf_ref[...] += jnp.dot(oht, m_fo, precision=lax.Precision.HIGHEST,
                             preferred_element_type=_f32)
    ones_col = jnp.ones((E_BLK, 1), _f32)
    cnt_ref[...] += jnp.dot(oht, ones_col, precision=lax.Precision.HIGHEST,
                            preferred_element_type=_f32)


def _run_kb(mid, epe, g, dst3, prev, b0, be0, feat, wmf_f, wcat, bias):
    kw = wcat.shape[1]
    prev_spec = (pl.BlockSpec((1, 1, E_BLK), lambda i: (i, 0, 0)) if mid
                 else pl.BlockSpec((E_BLK, F), lambda i: (i, 0)))
    out_specs = [
        pl.BlockSpec((E_BLK, F), lambda i: (i, 0)),
        pl.BlockSpec((E_BLK, F), lambda i: (i, 0)),
        pl.BlockSpec((E_BLK, F), lambda i: (i, 0)),
        pl.BlockSpec((F, F), lambda i: (0, 0)),
        pl.BlockSpec((F, 1), lambda i: (0, 0)),
    ]
    out_shape = [
        jax.ShapeDtypeStruct((E, F), _f32),
        jax.ShapeDtypeStruct((E, F), _f32),
        jax.ShapeDtypeStruct((E, F), _f32),
        jax.ShapeDtypeStruct((F, F), _f32),
        jax.ShapeDtypeStruct((F, 1), _f32),
    ]
    if not mid:
        out_specs = out_specs[1:2] + out_specs[3:]
        out_shape = out_shape[1:2] + out_shape[3:]
    res = pl.pallas_call(
        functools.partial(_kb_body, mid),
        grid=(NB,),
        in_specs=[
            pl.BlockSpec((E_BLK, F), lambda i: (i, 0)),
            pl.BlockSpec((E_BLK, 2 * F), lambda i: (i, 0)),
            pl.BlockSpec((1, 1, E_BLK), lambda i: (i, 0, 0)),
            prev_spec,
            pl.BlockSpec((F, F), lambda i: (0, 0)),
            pl.BlockSpec((1, F), lambda i: (0, 0)),
            pl.BlockSpec((F, F), lambda i: (0, 0)),
            pl.BlockSpec((F, F), lambda i: (0, 0)),
            pl.BlockSpec((F, kw), lambda i: (0, 0)),
            pl.BlockSpec((1, kw), lambda i: (0, 0)),
        ],
        out_specs=out_specs,
        out_shape=out_shape,
        compiler_params=pltpu.CompilerParams(
            dimension_semantics=("arbitrary",)),
    )(epe, g, dst3, prev, b0, be0, feat, wmf_f, wcat, bias)
    if mid:
        return res
    mof, aggf, cnt = res
    return None, mof, None, aggf, cnt


OB = 200


def _k6_body(node_ref, feat_ref, wo_ref, wfh_ref, bh_ref, wout_ref, bout_ref,
             w1_ref, b1_ref, w2_ref, b2_ref, dhat_ref, yhat_ref):
    obs_h = jnp.dot(node_ref[...], wo_ref[...], preferred_element_type=_f32)
    feat_h = jnp.dot(feat_ref[...], wfh_ref[...], preferred_element_type=_f32)
    h = jnp.maximum(obs_h[:, None, :] + feat_h[None, :, :]
                    + bh_ref[...][None, :, :], 0.0)
    dhat = jnp.sum(h * wout_ref[...][None, :, :], axis=2) + bout_ref[...]
    dhat_ref[...] = dhat
    hid = jnp.maximum(jnp.dot(dhat, w1_ref[...], preferred_element_type=_f32)
                      + b1_ref[...], 0.0)
    yhat_ref[...] = (jnp.dot(hid, w2_ref[...], preferred_element_type=_f32)
                     + b2_ref[...])


def _run_k6(node, feat, wo, wfh, bh, wout, bout, w1, b1, w2, b2):
    nblk = N_OBS // OB
    return pl.pallas_call(
        _k6_body,
        grid=(nblk,),
        in_specs=[
            pl.BlockSpec((OB, F), lambda i: (i, 0)),
            pl.BlockSpec((F, F), lambda i: (0, 0)),
            pl.BlockSpec((F, F), lambda i: (0, 0)),
            pl.BlockSpec((F, F), lambda i: (0, 0)),
            pl.BlockSpec((1, F), lambda i: (0, 0)),
            pl.BlockSpec((1, F), lambda i: (0, 0)),
            pl.BlockSpec((1, 1), lambda i: (0, 0)),
            pl.BlockSpec((F, F), lambda i: (0, 0)),
            pl.BlockSpec((1, F), lambda i: (0, 0)),
            pl.BlockSpec((F, 1), lambda i: (0, 0)),
            pl.BlockSpec((1, 1), lambda i: (0, 0)),
        ],
        out_specs=[
            pl.BlockSpec((OB, F), lambda i: (i, 0)),
            pl.BlockSpec((OB, 1), lambda i: (i, 0)),
        ],
        out_shape=[
            jax.ShapeDtypeStruct((N_OBS, F), _f32),
            jax.ShapeDtypeStruct((N_OBS, 1), _f32),
        ],
    )(node, feat, wo, wfh, bh, wout, bout, w1, b1, w2, b2)


@functools.lru_cache(maxsize=None)
def _sc_mesh():
    return plsc.VectorSubcoreMesh(core_axis_name="c", subcore_axis_name="s")


def _fill_vmem(ref, nrows, ncols, val):
    v = jnp.full((16,), val, _f32)

    def frow(r, _):
        def fcol(j, _):
            ref[r, pl.ds(j * 16, 16)] = v
            return 0
        return lax.fori_loop(0, ncols // 16, fcol, 0)
    lax.fori_loop(0, nrows, frow, 0)


RCH = 400
NRC = E_PER_W // RCH
SUB = RCH // CH
ZR = 80


def _scatter_body(with_count, mof_hbm, srcr_hbm, out_hbm, cnt_hbm,
                  rows_v, idx_v, zbuf_v, ones_v, cbuf_v, acc_sh, cacc_sh,
                  lsem, ssem, csem, zsem):
    cid = lax.axis_index("c")
    sid = lax.axis_index("s")
    wid = cid * NS + sid
    nrc = E_PER_W // RCH
    _fill_vmem(zbuf_v, ZR, F, 0.0)
    for p in range(RPAD // ZR):
        pltpu.async_copy(zbuf_v, acc_sh.at[pl.ds(sid * RPAD + p * ZR, ZR)],
                         zsem)
    if with_count:
        _fill_vmem(ones_v, CH, 16, 1.0)
        _fill_vmem(cbuf_v, ZR, 16, 0.0)
        for p in range(RPAD // ZR):
            pltpu.async_copy(cbuf_v,
                             cacc_sh.at[pl.ds(sid * RPAD + p * ZR, ZR)], zsem)
    for p in range(RPAD // ZR):
        pltpu.make_async_copy(
            zbuf_v, acc_sh.at[pl.ds(sid * RPAD + p * ZR, ZR)], zsem).wait()
        if with_count:
            pltpu.make_async_copy(
                cbuf_v, cacc_sh.at[pl.ds(sid * RPAD + p * ZR, ZR)],
                zsem).wait()
    pltpu.sync_copy(srcr_hbm.at[wid], idx_v)
    plsc.subcore_barrier()

    base = cid * E_PER_SC + sid * E_PER_W

    def mof_rows(co):
        return mof_hbm.at[pl.ds(base + co * RCH, RCH)]

    pltpu.async_copy(mof_rows(0), rows_v.at[0], lsem)

    def outer(co, _):
        b = co & 1
        pltpu.make_async_copy(mof_rows(co), rows_v.at[b], lsem).wait()

        @pl.when(co + 1 < nrc)
        def _():
            pltpu.async_copy(mof_rows(co + 1), rows_v.at[1 - b], lsem)
        for k in range(SUB):
            j = co * SUB + k
            pltpu.async_copy(rows_v.at[b, pl.ds(k * CH, CH)],
                             acc_sh.at[idx_v.at[j]], ssem, add=True)
            if with_count:
                pltpu.async_copy(ones_v, cacc_sh.at[idx_v.at[j]], csem,
                                 add=True)
        for k in range(SUB):
            pltpu.make_async_copy(rows_v.at[b, pl.ds(k * CH, CH)],
                                  acc_sh.at[idx_v.at[0]], ssem).wait()
            if with_count:
                pltpu.make_async_copy(ones_v, cacc_sh.at[idx_v.at[0]],
                                      csem).wait()
        return 0
    lax.fori_loop(0, nrc, outer, 0)
    plsc.subcore_barrier()
    for p in range(2):
        pltpu.sync_copy(acc_sh.at[pl.ds(sid * RPAD + p * 320, 320)],
                        rows_v.at[0, pl.ds(0, 320)])
        pltpu.sync_copy(rows_v.at[0, pl.ds(0, 320)],
                        out_hbm.at[pl.ds(cid * NPAD + sid * RPAD + p * 320,
                                         320)])
    if with_count:
        for p in range(RPAD // ZR):
            pltpu.sync_copy(cacc_sh.at[pl.ds(sid * RPAD + p * ZR, ZR)],
                            cbuf_v)
            pltpu.sync_copy(cbuf_v,
                            cnt_hbm.at[pl.ds(cid * NPAD + sid * RPAD + p * ZR,
                                             ZR)])


@functools.lru_cache(maxsize=None)
def _make_scatter(with_count):
    return pl.kernel(
        functools.partial(_scatter_body, with_count),
        out_type=[
            jax.ShapeDtypeStruct((NC * NPAD, F), _f32),
            jax.ShapeDtypeStruct((NC * NPAD, 16), _f32),
        ],
        mesh=_sc_mesh(),
        compiler_params=pltpu.CompilerParams(use_tc_tiling_on_sc=False),
        scratch_types=[
            pltpu.VMEM((2, RCH, F), _f32),
            pltpu.VMEM((NCH, CH), jnp.int32),
            pltpu.VMEM((ZR, F), _f32),
            pltpu.VMEM((CH, 16), _f32),
            pltpu.VMEM((ZR, 16), _f32),
            pltpu.VMEM_SHARED((NPAD, F), _f32),
            pltpu.VMEM_SHARED((NPAD, 16), _f32),
            pltpu.SemaphoreType.DMA,
            pltpu.SemaphoreType.DMA,
            pltpu.SemaphoreType.DMA,
            pltpu.SemaphoreType.DMA,
        ],
    )


def _sc_scatter(mof, srcr, with_count):
    out, cnt = _make_scatter(with_count)(mof, srcr)
    return out.reshape(NC, NPAD, F), cnt.reshape(NC, NPAD, 16)


def _gather_body(width, tab_hbm, srcr_hbm, out_hbm, idx_v, gb_v, gsem, wsem):
    cid = lax.axis_index("c")
    sid = lax.axis_index("s")
    wid = cid * NS + sid
    pltpu.sync_copy(srcr_hbm.at[wid], idx_v)
    base = cid * E_PER_SC + sid * E_PER_W

    pltpu.async_copy(tab_hbm.at[idx_v.at[0]], gb_v.at[0], gsem)

    def chunk(c, _):
        b = c & 1
        pltpu.make_async_copy(tab_hbm.at[idx_v.at[0]], gb_v.at[b],
                              gsem).wait()

        @pl.when(c >= 1)
        def _():
            pltpu.make_async_copy(gb_v.at[1 - b],
                                  out_hbm.at[pl.ds(base, CH)], wsem).wait()

        @pl.when(c + 1 < NCH)
        def _():
            pltpu.async_copy(tab_hbm.at[idx_v.at[c + 1]], gb_v.at[1 - b],
                             gsem)
        pltpu.async_copy(gb_v.at[b], out_hbm.at[pl.ds(base + c * CH, CH)],
                         wsem)
        return 0
    lax.fori_loop(0, NCH, chunk, 0)
    pltpu.make_async_copy(gb_v.at[0], out_hbm.at[pl.ds(base, CH)],
                          wsem).wait()


@functools.lru_cache(maxsize=None)
def _make_gather(width):
    return pl.kernel(
        functools.partial(_gather_body, width),
        out_type=jax.ShapeDtypeStruct((E, width), _f32),
        mesh=_sc_mesh(),
        compiler_params=pltpu.CompilerParams(use_tc_tiling_on_sc=False),
        scratch_types=[
            pltpu.VMEM((NCH, CH), jnp.int32),
            pltpu.VMEM((2, CH, width), _f32),
            pltpu.SemaphoreType.DMA,
            pltpu.SemaphoreType.DMA,
        ],
    )


def _sc_gather(tab, srcr):
    return _make_gather(tab.shape[1])(tab, srcr)


def kernel(x, edge_index, edge_value, params):
    src = edge_index[0].astype(jnp.int32)
    dst3 = edge_index[1].astype(_f32).reshape(NB, 1, E_BLK)
    ev3 = edge_value.reshape(NB, 1, E_BLK)
    n = N_OBS

    node_emb = jnp.ones((n, F), _f32)
    feature_emb = jnp.eye(F, dtype=_f32)

    def split(p, ein):
        wmf_f, wmf_e = p['Wmf'][:F], p['Wmf'][F:]
        wmo_n, wmo_e = p['Wmo'][:F], p['Wmo'][F:]
        we_e = p['We'][:ein]
        we_n = p['We'][ein:ein + F]
        we_f = p['We'][ein + F:]
        wcat = jnp.concatenate([wmf_e, wmo_e, we_e], axis=1)
        bias = jnp.concatenate([p['bmf'], p['bmo'], p['be']]).reshape(1, 3 * F)
        return wmf_f, wmo_n, wcat, bias, we_n, we_f

    blocks = [params['block%d' % i] for i in range(NUM_LAYERS)]
    sp = [split(blocks[i], 1 if i == 0 else F) for i in range(NUM_LAYERS)]

    srcr = src.reshape(NC * NS, NCH, CH)
    wmf_f0, wmo_n0, wcat0, bias0, we_n0, we_f0 = sp[0]
    wmf_f1, wmo_n1, wcat1, bias1, we_n1, we_f1 = sp[1]
    wmf_f2, wmo_n2, wcat2, bias2, _, _ = sp[2]

    psrc0 = jnp.sum(wmo_n0, axis=0, keepdims=True)
    mof0, epe0, aggf0, cntd = _run_k1(
        ev3, dst3, psrc0, wmf_f0, wcat0, bias0)
    aggo0, cnt_parts = _sc_scatter(mof0, srcr, with_count=True)
    node1, apn0, feat1, b0 = _run_k3(
        False, node_emb, aggo0, cnt_parts, aggf0, cntd, feature_emb,
        blocks[0]['Wn'][:F], blocks[0]['Wn'][F:], blocks[0]['bn'].reshape(1, F),
        blocks[0]['Wf'][:F], blocks[0]['Wf'][F:], blocks[0]['bf'].reshape(1, F),
        we_n0, we_f0, wmo_n1)
    g0 = _sc_gather(apn0, srcr)

    edge1, mof1, epe1, aggf1, _ = _run_kb(
        True, epe0, g0, dst3, ev3, b0, bias0[:, 2 * F:],
        feat1, wmf_f1, wcat1, bias1)
    aggo1, _ = _sc_scatter(mof1, srcr, with_count=False)
    node2, apn1, feat2, b1 = _run_k3(
        False, node1, aggo1, cnt_parts, aggf1, cntd, feat1,
        blocks[1]['Wn'][:F], blocks[1]['Wn'][F:], blocks[1]['bn'].reshape(1, F),
        blocks[1]['Wf'][:F], blocks[1]['Wf'][F:], blocks[1]['bf'].reshape(1, F),
        we_n1, we_f1, wmo_n2)
    g1 = _sc_gather(apn1, srcr)

    _, mof2, _, aggf2, _ = _run_kb(
        False, epe1, g1, dst3, edge1, b1, bias1[:, 2 * F:],
        feat2, wmf_f2, wcat2[:, :2 * F], bias2[:, :2 * F])
    aggo2, _ = _sc_scatter(mof2, srcr, with_count=False)
    node3, feat3 = _run_k3(
        True, node2, aggo2, cnt_parts, aggf2, cntd, feat2,
        blocks[2]['Wn'][:F], blocks[2]['Wn'][F:], blocks[2]['bn'].reshape(1, F),
        blocks[2]['Wf'][:F], blocks[2]['Wf'][F:], blocks[2]['bf'].reshape(1, F),
        we_n0, we_f0, wmo_n0)
    node_emb = node3
    feature_emb = feat3

    ep = params['eph']
    npar = params['nph']
    d_hat, y_hat = _run_k6(
        node_emb, feature_emb, ep['Wo'], ep['Wf'], ep['bh'].reshape(1, F),
        ep['wout'].reshape(1, F), ep['bout'].reshape(1, 1),
        npar['W1'], npar['b1'].reshape(1, F),
        npar['W2'], npar['b2'].reshape(1, 1))
    return d_hat, y_hat


# speedup vs baseline: 1.7151x; 1.7151x over previous
import functools

import jax
import jax.numpy as jnp
from jax import lax
from jax.experimental import pallas as pl
from jax.experimental.pallas import tpu as pltpu
from jax.experimental.pallas import tpu_sc as plsc

N_OBS = 10000
F = 64
E = 320000
NUM_LAYERS = 3

E_BLK = 5000
NB = E // E_BLK

NC = 2
NS = 16
E_PER_SC = E // NC
E_PER_W = E_PER_SC // NS
CH = 80
NCH = E_PER_W // CH
RPAD = 640
NPAD = NS * RPAD

_f32 = jnp.float32


def _k1_body(evh_ref, evl_ref, dst_ref, psrc_ref, wmf_f_ref,
             wcat_ref, bias_ref, mof_ref, epe_ref, aggf_ref, cnt_ref):
    ones_row = jnp.ones((1, 1), _f32)
    dims = (((0,), (0,)), ((), ()))
    ev_col = (lax.dot_general(evh_ref[0], ones_row, dims,
                              preferred_element_type=_f32)
              + lax.dot_general(evl_ref[0], ones_row, dims,
                                preferred_element_type=_f32))
    iota_col = lax.broadcasted_iota(jnp.int32, (F, 1), 0).astype(_f32)
    oht = (iota_col == dst_ref[0]).astype(_f32)
    oh = lax.dot_general(oht, jnp.eye(F, dtype=_f32), dims,
                         preferred_element_type=_f32)
    ep = ev_col * wcat_ref[...]
    m_of = jnp.maximum(jnp.dot(oh, wmf_f_ref[...], preferred_element_type=_f32)
                       + ep[:, :F] + bias_ref[:, :F], 0.0)
    mof_ref[...] = m_of
    m_fo = jnp.maximum(psrc_ref[...] + ep[:, F:2 * F]
                       + bias_ref[:, F:2 * F], 0.0)
    epe_ref[...] = ep[:, 2 * F:]

    @pl.when(pl.program_id(0) == 0)
    def _():
        aggf_ref[...] = jnp.zeros_like(aggf_ref)
        cnt_ref[...] = jnp.zeros_like(cnt_ref)

    aggf_ref[...] += lax.dot_general(oh, m_fo, dims,
                                     preferred_element_type=_f32)
    cnt_ref[...] += jnp.sum(oht, axis=1, keepdims=True)


def _run_k1(evh3, evl3, dst3, psrc, wmf_f, wcat, bias):
    return pl.pallas_call(
        _k1_body,
        grid=(NB,),
        in_specs=[
            pl.BlockSpec((1, 1, E_BLK), lambda i: (i, 0, 0)),
            pl.BlockSpec((1, 1, E_BLK), lambda i: (i, 0, 0)),
            pl.BlockSpec((1, 1, E_BLK), lambda i: (i, 0, 0)),
            pl.BlockSpec((1, F), lambda i: (0, 0)),
            pl.BlockSpec((F, F), lambda i: (0, 0)),
            pl.BlockSpec((1, 3 * F), lambda i: (0, 0)),
            pl.BlockSpec((1, 3 * F), lambda i: (0, 0)),
        ],
        out_specs=[
            pl.BlockSpec((E_BLK, F), lambda i: (i, 0)),
            pl.BlockSpec((E_BLK, F), lambda i: (i, 0)),
            pl.BlockSpec((F, F), lambda i: (0, 0)),
            pl.BlockSpec((F, 1), lambda i: (0, 0)),
        ],
        out_shape=[
            jax.ShapeDtypeStruct((E, F), _f32),
            jax.ShapeDtypeStruct((E, F), _f32),
            jax.ShapeDtypeStruct((F, F), _f32),
            jax.ShapeDtypeStruct((F, 1), _f32),
        ],
        compiler_params=pltpu.CompilerParams(
            dimension_semantics=("arbitrary",)),
    )(evh3, evl3, dst3, psrc, wmf_f, wcat, bias)


def _k3_body(last, node_ref, aggo_ref, cntp_ref, aggf_ref, cntd_ref,
             feat_ref, wn_n_ref, wn_a_ref, bn_ref, wf_f_ref, wf_a_ref,
             bf_ref, we_n_ref, we_f_ref, wmo_nx_ref,
             node2_ref, *out_refs):
    if last:
        feat2_ref, = out_refs
    else:
        a_ref, feat2_ref, b_ref = out_refs
    node = node_ref[...]
    cnt = jnp.maximum(cntp_ref[0][:N_OBS, 0:1] + cntp_ref[1][:N_OBS, 0:1], 1.0)
    aggo = (aggo_ref[0][:N_OBS] + aggo_ref[1][:N_OBS]) / cnt
    new_node = jnp.maximum(
        jnp.dot(node, wn_n_ref[...], preferred_element_type=_f32)
        + jnp.dot(aggo, wn_a_ref[...], preferred_element_type=_f32)
        + bn_ref[...], 0.0)
    node2 = jnp.maximum(new_node + node, 0.0)
    node2_ref[...] = node2
    if not last:
        a = jnp.dot(new_node, we_n_ref[...], preferred_element_type=_f32)
        pn = jnp.dot(node2, wmo_nx_ref[...], preferred_element_type=_f32)
        a_ref[...] = jnp.concatenate([a, pn], axis=1)
    feat = feat_ref[...]
    aggf = aggf_ref[...] / jnp.maximum(cntd_ref[...], 1.0)
    new_feat = jnp.maximum(
        jnp.dot(feat, wf_f_ref[...], preferred_element_type=_f32)
        + jnp.dot(aggf, wf_a_ref[...], preferred_element_type=_f32)
        + bf_ref[...], 0.0)
    feat2 = jnp.maximum(new_feat + feat, 0.0)
    feat2_ref[...] = feat2
    if not last:
        b_ref[...] = jnp.dot(new_feat, we_f_ref[...],
                             preferred_element_type=_f32)


def _run_k3(last, node, aggo_parts, cnt_parts, aggf, cntd, feat,
            wn_n, wn_a, bn, wf_f, wf_a, bf, we_n, we_f, wmo_nx):
    return pl.pallas_call(
        functools.partial(_k3_body, last),
        out_shape=([jax.ShapeDtypeStruct((N_OBS, F), _f32),
                    jax.ShapeDtypeStruct((F, F), _f32)] if last else
                   [jax.ShapeDtypeStruct((N_OBS, F), _f32),
                    jax.ShapeDtypeStruct((N_OBS, 2 * F), _f32),
                    jax.ShapeDtypeStruct((F, F), _f32),
                    jax.ShapeDtypeStruct((F, F), _f32)]),
    )(node, aggo_parts, cnt_parts, aggf, cntd, feat,
      wn_n, wn_a, bn, wf_f, wf_a, bf, we_n, we_f, wmo_nx)


def _kb_body(mid, *refs):
    (epe_ref, g_ref, dst_ref, prevh_ref, prevl_ref, b0_ref, be0_ref,
     feat_ref, wmf_f_ref, wcat_ref, bias_ref) = refs[:11]
    if mid:
        edge_ref, mof_ref, epe_out_ref, aggf_ref, cnt_ref = refs[11:]
    else:
        mof_ref, aggf_ref, cnt_ref = refs[11:]
    dims = (((0,), (0,)), ((), ()))
    iota_col = lax.broadcasted_iota(jnp.int32, (F, 1), 0).astype(_f32)
    oht = (iota_col == dst_ref[0]).astype(_f32)
    oh = lax.dot_general(oht, jnp.eye(F, dtype=_f32), dims,
                         preferred_element_type=_f32)
    g = g_ref[...]
    new_edge = jnp.maximum(
        epe_ref[...] + g[:, :F]
        + jnp.dot(oh, b0_ref[...], preferred_element_type=_f32)
        + be0_ref[...], 0.0)
    if mid:
        ones_row = jnp.ones((1, 1), _f32)
        prev = (lax.dot_general(prevh_ref[0], ones_row, dims,
                                preferred_element_type=_f32)
                + lax.dot_general(prevl_ref[0], ones_row, dims,
                                  preferred_element_type=_f32))
    else:
        prev = prevh_ref[...]
    edge = jnp.maximum(new_edge + prev, 0.0)
    if mid:
        edge_ref[...] = edge
    ep = jnp.dot(edge, wcat_ref[...], preferred_element_type=_f32)
    t = jnp.dot(feat_ref[...], wmf_f_ref[...], preferred_element_type=_f32)
    m_of = jnp.maximum(jnp.dot(oh, t, preferred_element_type=_f32)
                       + ep[:, :F] + bias_ref[:, :F], 0.0)
    mof_ref[...] = m_of
    m_fo = jnp.maximum(g[:, F:2 * F] + ep[:, F:2 * F]
                       + bias_ref[:, F:2 * F], 0.0)
    if mid:
        epe_out_ref[...] = ep[:, 2 * F:]

    @pl.when(pl.program_id(0) == 0)
    def _():
        aggf_ref[...] = jnp.zeros_like(aggf_ref)
        cnt_ref[...] = jnp.zeros_like(cnt_ref)

    aggf_ref[...] += lax.dot_general(oh, m_fo, dims,
                                     preferred_element_type=_f32)
    cnt_ref[...] += jnp.sum(oht, axis=1, keepdims=True)


def _run_kb(mid, epe, g, dst3, prevh, prevl, b0, be0, feat, wmf_f,
            wcat, bias):
    kw = wcat.shape[1]
    if mid:
        prevh_spec = pl.BlockSpec((1, 1, E_BLK), lambda i: (i, 0, 0))
        prevl_spec = pl.BlockSpec((1, 1, E_BLK), lambda i: (i, 0, 0))
    else:
        prevh_spec = pl.BlockSpec((E_BLK, F), lambda i: (i, 0))
        prevl_spec = pl.BlockSpec((1, 1), lambda i: (0, 0))
    out_specs = [
        pl.BlockSpec((E_BLK, F), lambda i: (i, 0)),
        pl.BlockSpec((E_BLK, F), lambda i: (i, 0)),
        pl.BlockSpec((E_BLK, F), lambda i: (i, 0)),
        pl.BlockSpec((F, F), lambda i: (0, 0)),
        pl.BlockSpec((F, 1), lambda i: (0, 0)),
    ]
    out_shape = [
        jax.ShapeDtypeStruct((E, F), _f32),
        jax.ShapeDtypeStruct((E, F), _f32),
        jax.ShapeDtypeStruct((E, F), _f32),
        jax.ShapeDtypeStruct((F, F), _f32),
        jax.ShapeDtypeStruct((F, 1), _f32),
    ]
    if not mid:
        out_specs = out_specs[1:2] + out_specs[3:]
        out_shape = out_shape[1:2] + out_shape[3:]
    res = pl.pallas_call(
        functools.partial(_kb_body, mid),
        grid=(NB,),
        in_specs=[
            pl.BlockSpec((E_BLK, F), lambda i: (i, 0)),
            pl.BlockSpec((E_BLK, 2 * F), lambda i: (i, 0)),
            pl.BlockSpec((1, 1, E_BLK), lambda i: (i, 0, 0)),
            prevh_spec,
            prevl_spec,
            pl.BlockSpec((F, F), lambda i: (0, 0)),
            pl.BlockSpec((1, F), lambda i: (0, 0)),
            pl.BlockSpec((F, F), lambda i: (0, 0)),
            pl.BlockSpec((F, F), lambda i: (0, 0)),
            pl.BlockSpec((F, kw), lambda i: (0, 0)),
            pl.BlockSpec((1, kw), lambda i: (0, 0)),
        ],
        out_specs=out_specs,
        out_shape=out_shape,
        compiler_params=pltpu.CompilerParams(
            dimension_semantics=("arbitrary",)),
    )(epe, g, dst3, prevh, prevl, b0, be0, feat, wmf_f, wcat, bias)
    if mid:
        return res
    mof, aggf, cnt = res
    return None, mof, None, aggf, cnt


OB = 200


def _k6_body(node_ref, feat_ref, wo_ref, wfh_ref, bh_ref, wout_ref, bout_ref,
             w1_ref, b1_ref, w2_ref, b2_ref, dhat_ref, yhat_ref):
    obs_h = jnp.dot(node_ref[...], wo_ref[...], preferred_element_type=_f32)
    feat_h = jnp.dot(feat_ref[...], wfh_ref[...], preferred_element_type=_f32)
    h = jnp.maximum(obs_h[:, None, :] + feat_h[None, :, :]
                    + bh_ref[...][None, :, :], 0.0)
    dhat = jnp.sum(h * wout_ref[...][None, :, :], axis=2) + bout_ref[...]
    dhat_ref[...] = dhat
    hid = jnp.maximum(jnp.dot(dhat, w1_ref[...], preferred_element_type=_f32)
                      + b1_ref[...], 0.0)
    yhat_ref[...] = (jnp.dot(hid, w2_ref[...], preferred_element_type=_f32)
                     + b2_ref[...])


def _run_k6(node, feat, wo, wfh, bh, wout, bout, w1, b1, w2, b2):
    nblk = N_OBS // OB
    return pl.pallas_call(
        _k6_body,
        grid=(nblk,),
        in_specs=[
            pl.BlockSpec((OB, F), lambda i: (i, 0)),
            pl.BlockSpec((F, F), lambda i: (0, 0)),
            pl.BlockSpec((F, F), lambda i: (0, 0)),
            pl.BlockSpec((F, F), lambda i: (0, 0)),
            pl.BlockSpec((1, F), lambda i: (0, 0)),
            pl.BlockSpec((1, F), lambda i: (0, 0)),
            pl.BlockSpec((1, 1), lambda i: (0, 0)),
            pl.BlockSpec((F, F), lambda i: (0, 0)),
            pl.BlockSpec((1, F), lambda i: (0, 0)),
            pl.BlockSpec((F, 1), lambda i: (0, 0)),
            pl.BlockSpec((1, 1), lambda i: (0, 0)),
        ],
        out_specs=[
            pl.BlockSpec((OB, F), lambda i: (i, 0)),
            pl.BlockSpec((OB, 1), lambda i: (i, 0)),
        ],
        out_shape=[
            jax.ShapeDtypeStruct((N_OBS, F), _f32),
            jax.ShapeDtypeStruct((N_OBS, 1), _f32),
        ],
    )(node, feat, wo, wfh, bh, wout, bout, w1, b1, w2, b2)


@functools.lru_cache(maxsize=None)
def _sc_mesh():
    return plsc.VectorSubcoreMesh(core_axis_name="c", subcore_axis_name="s")


def _fill_vmem(ref, nrows, ncols, val):
    v = jnp.full((16,), val, _f32)

    def frow(r, _):
        def fcol(j, _):
            ref[r, pl.ds(j * 16, 16)] = v
            return 0
        return lax.fori_loop(0, ncols // 16, fcol, 0)
    lax.fori_loop(0, nrows, frow, 0)


RCH = 400
NRC = E_PER_W // RCH
SUB = RCH // CH
ZR = 80


def _scatter_body(with_count, mof_hbm, srcr_hbm, out_hbm, cnt_hbm,
                  rows_v, idx_v, zbuf_v, ones_v, cbuf_v, acc_sh, cacc_sh,
                  lsem, ssem, csem, zsem):
    cid = lax.axis_index("c")
    sid = lax.axis_index("s")
    wid = cid * NS + sid
    nrc = E_PER_W // RCH
    _fill_vmem(zbuf_v, ZR, F, 0.0)
    for p in range(RPAD // ZR):
        pltpu.async_copy(zbuf_v, acc_sh.at[pl.ds(sid * RPAD + p * ZR, ZR)],
                         zsem)
    if with_count:
        _fill_vmem(ones_v, CH, 16, 1.0)
        _fill_vmem(cbuf_v, ZR, 16, 0.0)
        for p in range(RPAD // ZR):
            pltpu.async_copy(cbuf_v,
                             cacc_sh.at[pl.ds(sid * RPAD + p * ZR, ZR)], zsem)
    for p in range(RPAD // ZR):
        pltpu.make_async_copy(
            zbuf_v, acc_sh.at[pl.ds(sid * RPAD + p * ZR, ZR)], zsem).wait()
        if with_count:
            pltpu.make_async_copy(
                cbuf_v, cacc_sh.at[pl.ds(sid * RPAD + p * ZR, ZR)],
                zsem).wait()
    pltpu.sync_copy(srcr_hbm.at[wid], idx_v)
    plsc.subcore_barrier()

    base = cid * E_PER_SC + sid * E_PER_W

    def mof_rows(co):
        return mof_hbm.at[pl.ds(base + co * RCH, RCH)]

    pltpu.async_copy(mof_rows(0), rows_v.at[0], lsem)

    def outer(co, _):
        b = co & 1
        pltpu.make_async_copy(mof_rows(co), rows_v.at[b], lsem).wait()

        @pl.when(co + 1 < nrc)
        def _():
            pltpu.async_copy(mof_rows(co + 1), rows_v.at[1 - b], lsem)
        for k in range(SUB):
            j = co * SUB + k
            pltpu.async_copy(rows_v.at[b, pl.ds(k * CH, CH)],
                             acc_sh.at[idx_v.at[j]], ssem, add=True)
            if with_count:
                pltpu.async_copy(ones_v, cacc_sh.at[idx_v.at[j]], csem,
                                 add=True)
        for k in range(SUB):
            pltpu.make_async_copy(rows_v.at[b, pl.ds(k * CH, CH)],
                                  acc_sh.at[idx_v.at[0]], ssem).wait()
            if with_count:
                pltpu.make_async_copy(ones_v, cacc_sh.at[idx_v.at[0]],
                                      csem).wait()
        return 0
    lax.fori_loop(0, nrc, outer, 0)
    plsc.subcore_barrier()
    for p in range(2):
        pltpu.sync_copy(acc_sh.at[pl.ds(sid * RPAD + p * 320, 320)],
                        rows_v.at[0, pl.ds(0, 320)])
        pltpu.sync_copy(rows_v.at[0, pl.ds(0, 320)],
                        out_hbm.at[pl.ds(cid * NPAD + sid * RPAD + p * 320,
                                         320)])
    if with_count:
        for p in range(RPAD // ZR):
            pltpu.sync_copy(cacc_sh.at[pl.ds(sid * RPAD + p * ZR, ZR)],
                            cbuf_v)
            pltpu.sync_copy(cbuf_v,
                            cnt_hbm.at[pl.ds(cid * NPAD + sid * RPAD + p * ZR,
                                             ZR)])


@functools.lru_cache(maxsize=None)
def _make_scatter(with_count):
    return pl.kernel(
        functools.partial(_scatter_body, with_count),
        out_type=[
            jax.ShapeDtypeStruct((NC * NPAD, F), _f32),
            jax.ShapeDtypeStruct((NC * NPAD, 16), _f32),
        ],
        mesh=_sc_mesh(),
        compiler_params=pltpu.CompilerParams(use_tc_tiling_on_sc=False),
        scratch_types=[
            pltpu.VMEM((2, RCH, F), _f32),
            pltpu.VMEM((NCH, CH), jnp.int32),
            pltpu.VMEM((ZR, F), _f32),
            pltpu.VMEM((CH, 16), _f32),
            pltpu.VMEM((ZR, 16), _f32),
            pltpu.VMEM_SHARED((NPAD, F), _f32),
            pltpu.VMEM_SHARED((NPAD, 16), _f32),
            pltpu.SemaphoreType.DMA,
            pltpu.SemaphoreType.DMA,
            pltpu.SemaphoreType.DMA,
            pltpu.SemaphoreType.DMA,
        ],
    )


def _sc_scatter(mof, srcr, with_count):
    out, cnt = _make_scatter(with_count)(mof, srcr)
    return out.reshape(NC, NPAD, F), cnt.reshape(NC, NPAD, 16)


def _gather_body(width, tab_hbm, srcr_hbm, out_hbm, idx_v, gb_v, gsem, wsem):
    cid = lax.axis_index("c")
    sid = lax.axis_index("s")
    wid = cid * NS + sid
    pltpu.sync_copy(srcr_hbm.at[wid], idx_v)
    base = cid * E_PER_SC + sid * E_PER_W

    pltpu.async_copy(tab_hbm.at[idx_v.at[0]], gb_v.at[0], gsem)

    def chunk(c, _):
        b = c & 1
        pltpu.make_async_copy(tab_hbm.at[idx_v.at[0]], gb_v.at[b],
                              gsem).wait()

        @pl.when(c >= 1)
        def _():
            pltpu.make_async_copy(gb_v.at[1 - b],
                                  out_hbm.at[pl.ds(base, CH)], wsem).wait()

        @pl.when(c + 1 < NCH)
        def _():
            pltpu.async_copy(tab_hbm.at[idx_v.at[c + 1]], gb_v.at[1 - b],
                             gsem)
        pltpu.async_copy(gb_v.at[b], out_hbm.at[pl.ds(base + c * CH, CH)],
                         wsem)
        return 0
    lax.fori_loop(0, NCH, chunk, 0)
    pltpu.make_async_copy(gb_v.at[0], out_hbm.at[pl.ds(base, CH)],
                          wsem).wait()


@functools.lru_cache(maxsize=None)
def _make_gather(width):
    return pl.kernel(
        functools.partial(_gather_body, width),
        out_type=jax.ShapeDtypeStruct((E, width), _f32),
        mesh=_sc_mesh(),
        compiler_params=pltpu.CompilerParams(use_tc_tiling_on_sc=False),
        scratch_types=[
            pltpu.VMEM((NCH, CH), jnp.int32),
            pltpu.VMEM((2, CH, width), _f32),
            pltpu.SemaphoreType.DMA,
            pltpu.SemaphoreType.DMA,
        ],
    )


def _sc_gather(tab, srcr):
    return _make_gather(tab.shape[1])(tab, srcr)


def kernel(x, edge_index, edge_value, params):
    src = edge_index[0].astype(jnp.int32)
    dst3 = edge_index[1].astype(_f32).reshape(NB, 1, E_BLK)
    evh = edge_value.astype(jnp.bfloat16).astype(_f32)
    evh3 = evh.reshape(NB, 1, E_BLK)
    evl3 = (edge_value - evh).reshape(NB, 1, E_BLK)
    n = N_OBS

    node_emb = jnp.ones((n, F), _f32)
    feature_emb = jnp.eye(F, dtype=_f32)

    def split(p, ein):
        wmf_f, wmf_e = p['Wmf'][:F], p['Wmf'][F:]
        wmo_n, wmo_e = p['Wmo'][:F], p['Wmo'][F:]
        we_e = p['We'][:ein]
        we_n = p['We'][ein:ein + F]
        we_f = p['We'][ein + F:]
        wcat = jnp.concatenate([wmf_e, wmo_e, we_e], axis=1)
        bias = jnp.concatenate([p['bmf'], p['bmo'], p['be']]).reshape(1, 3 * F)
        return wmf_f, wmo_n, wcat, bias, we_n, we_f

    blocks = [params['block%d' % i] for i in range(NUM_LAYERS)]
    sp = [split(blocks[i], 1 if i == 0 else F) for i in range(NUM_LAYERS)]

    srcr = src.reshape(NC * NS, NCH, CH)
    wmf_f0, wmo_n0, wcat0, bias0, we_n0, we_f0 = sp[0]
    wmf_f1, wmo_n1, wcat1, bias1, we_n1, we_f1 = sp[1]
    wmf_f2, wmo_n2, wcat2, bias2, _, _ = sp[2]

    psrc0 = jnp.sum(wmo_n0, axis=0, keepdims=True)
    mof0, epe0, aggf0, cntd = _run_k1(
        evh3, evl3, dst3, psrc0, wmf_f0, wcat0, bias0)
    aggo0, cnt_parts = _sc_scatter(mof0, srcr, with_count=True)
    node1, apn0, feat1, b0 = _run_k3(
        False, node_emb, aggo0, cnt_parts, aggf0, cntd, feature_emb,
        blocks[0]['Wn'][:F], blocks[0]['Wn'][F:], blocks[0]['bn'].reshape(1, F),
        blocks[0]['Wf'][:F], blocks[0]['Wf'][F:], blocks[0]['bf'].reshape(1, F),
        we_n0, we_f0, wmo_n1)
    g0 = _sc_gather(apn0, srcr)

    edge1, mof1, epe1, aggf1, _ = _run_kb(
        True, epe0, g0, dst3, evh3, evl3, b0, bias0[:, 2 * F:],
        feat1, wmf_f1, wcat1, bias1)
    aggo1, _ = _sc_scatter(mof1, srcr, with_count=False)
    node2, apn1, feat2, b1 = _run_k3(
        False, node1, aggo1, cnt_parts, aggf1, cntd, feat1,
        blocks[1]['Wn'][:F], blocks[1]['Wn'][F:], blocks[1]['bn'].reshape(1, F),
        blocks[1]['Wf'][:F], blocks[1]['Wf'][F:], blocks[1]['bf'].reshape(1, F),
        we_n1, we_f1, wmo_n2)
    g1 = _sc_gather(apn1, srcr)

    dummy = jnp.zeros((1, 1), _f32)
    _, mof2, _, aggf2, _ = _run_kb(
        False, epe1, g1, dst3, edge1, dummy, b1, bias1[:, 2 * F:],
        feat2, wmf_f2, wcat2[:, :2 * F], bias2[:, :2 * F])
    aggo2, _ = _sc_scatter(mof2, srcr, with_count=False)
    node3, feat3 = _run_k3(
        True, node2, aggo2, cnt_parts, aggf2, cntd, feat2,
        blocks[2]['Wn'][:F], blocks[2]['Wn'][F:], blocks[2]['bn'].reshape(1, F),
        blocks[2]['Wf'][:F], blocks[2]['Wf'][F:], blocks[2]['bf'].reshape(1, F),
        we_n0, we_f0, wmo_n0)
    node_emb = node3
    feature_emb = feat3

    ep = params['eph']
    npar = params['nph']
    d_hat, y_hat = _run_k6(
        node_emb, feature_emb, ep['Wo'], ep['Wf'], ep['bh'].reshape(1, F),
        ep['wout'].reshape(1, F), ep['bout'].reshape(1, 1),
        npar['W1'], npar['b1'].reshape(1, F),
        npar['W2'], npar['b2'].reshape(1, 1))
    return d_hat, y_hat
